# Initial kernel scaffold; baseline (speedup 1.0000x reference)
import jax
import jax.numpy as jnp
from jax.experimental import pallas as pl


def kernel(x, edge_index, W1, b1, W2, b2, W3, b3, W4, b4):
    raise NotImplementedError("write your pallas kernel here")


# speedup vs baseline: 12.2429x; 12.2429x over previous
import functools

import jax
import jax.numpy as jnp
from jax import lax
from jax.experimental import pallas as pl
from jax.experimental.pallas import tpu as pltpu
from jax.experimental.pallas import tpu_sc as plsc

_NC = 2
_NS = 16
_NW = _NC * _NS
_L = 16


def _mesh():
    return plsc.VectorSubcoreMesh(
        core_axis_name="c", subcore_axis_name="s",
        num_cores=_NC, num_subcores=_NS)


_SC_PARAMS = pltpu.CompilerParams(needs_layout_passes=False)


def _wid():
    return lax.axis_index("s") * _NC + lax.axis_index("c")


def _zero_loop(refs, n):
    zeros = jnp.zeros((_L,), jnp.float32)

    def body(i, carry):
        for r in refs:
            r[pl.ds(i * _L, _L)] = zeros
        return carry

    lax.fori_loop(0, n // _L, body, 0)


@functools.lru_cache(maxsize=None)
def _make_count(n, e):
    e_per = e // _NW

    @functools.partial(
        pl.kernel,
        mesh=_mesh(),
        compiler_params=_SC_PARAMS,
        out_type=jax.ShapeDtypeStruct((_NW, n), jnp.float32),
        scratch_types=[
            pltpu.VMEM((e_per,), jnp.int32),
            pltpu.VMEM((n,), jnp.float32),
        ],
    )
    def count(col_hbm, out_hbm, cb, ob):
        w = _wid()
        pltpu.sync_copy(col_hbm.at[pl.ds(w * e_per, e_per)], cb)
        _zero_loop([ob], n)
        ones = jnp.ones((_L,), jnp.float32)

        def step(j, carry):
            idx = cb[pl.ds(j * _L, _L)]
            plsc.addupdate_scatter(ob, [idx], ones)
            return carry

        lax.fori_loop(0, e_per // _L, step, 0)
        pltpu.sync_copy(ob, out_hbm.at[w])

    return count


@functools.lru_cache(maxsize=None)
def _make_prop1(n, e):
    e_per = e // _NW

    @functools.partial(
        pl.kernel,
        mesh=_mesh(),
        compiler_params=_SC_PARAMS,
        out_type=jax.ShapeDtypeStruct((_NW, n), jnp.float32),
        scratch_types=[
            pltpu.VMEM((n,), jnp.float32),
            pltpu.VMEM((e_per,), jnp.int32),
            pltpu.VMEM((e_per,), jnp.int32),
            pltpu.VMEM((n,), jnp.float32),
        ],
    )
    def prop1(y_hbm, row_hbm, col_hbm, out_hbm, yv, rb, cb, ob):
        w = _wid()
        pltpu.sync_copy(y_hbm.at[0], yv)
        pltpu.sync_copy(row_hbm.at[pl.ds(w * e_per, e_per)], rb)
        pltpu.sync_copy(col_hbm.at[pl.ds(w * e_per, e_per)], cb)
        _zero_loop([ob], n)

        def step(j, carry):
            r = rb[pl.ds(j * _L, _L)]
            c = cb[pl.ds(j * _L, _L)]
            v = plsc.load_gather(yv, [r])
            plsc.addupdate_scatter(ob, [c], v)
            return carry

        lax.fori_loop(0, e_per // _L, step, 0)
        pltpu.sync_copy(ob, out_hbm.at[w])

    return prop1


@functools.lru_cache(maxsize=None)
def _make_prop(n, e, dfull, chunk):
    d = dfull // _NW
    nchunks = e // chunk
    scratch = (
        [pltpu.VMEM((n,), jnp.float32) for _ in range(2 * d)]
        + [pltpu.VMEM((chunk,), jnp.int32) for _ in range(2)]
    )

    @functools.partial(
        pl.kernel,
        mesh=_mesh(),
        compiler_params=_SC_PARAMS,
        out_type=jax.ShapeDtypeStruct((dfull, n), jnp.float32),
        scratch_types=scratch,
    )
    def prop(y_hbm, row_hbm, col_hbm, out_hbm, *bufs):
        yb = bufs[:d]
        zb = bufs[d:2 * d]
        rb = bufs[2 * d]
        cb = bufs[2 * d + 1]
        w = _wid()
        f0 = w * d
        for f in range(d):
            pltpu.sync_copy(y_hbm.at[f0 + f], yb[f])
        _zero_loop(list(zb), n)

        def chunk_body(k, carry):
            pltpu.sync_copy(row_hbm.at[pl.ds(k * chunk, chunk)], rb)
            pltpu.sync_copy(col_hbm.at[pl.ds(k * chunk, chunk)], cb)

            def step(j, c2):
                r = rb[pl.ds(j * _L, _L)]
                c = cb[pl.ds(j * _L, _L)]
                for f in range(d):
                    v = plsc.load_gather(yb[f], [r])
                    plsc.addupdate_scatter(zb[f], [c], v)
                return c2

            lax.fori_loop(0, chunk // _L, step, 0)
            return carry

        lax.fori_loop(0, nchunks, chunk_body, 0)
        for f in range(d):
            pltpu.sync_copy(zb[f], out_hbm.at[f0 + f])

    return prop


def _stage1_body(w1_ref, x_ref, cnt_ref, yt_ref, dinv_ref):
    deg = 1.0 + jnp.sum(cnt_ref[...], axis=0, keepdims=True)
    dinv = lax.rsqrt(deg)
    dinv_ref[...] = dinv
    yt_ref[...] = dinv * lax.dot_general(
        w1_ref[...], x_ref[...], (((0,), (1,)), ((), ())),
        preferred_element_type=jnp.float32)


def _mid_body(z_ref, y_ref, dinv_ref, b_ref, w_ref, out_ref):
    dinv = dinv_ref[...]
    h = jnp.tanh(dinv * (z_ref[...] + y_ref[...]) + b_ref[...])
    out_ref[...] = dinv * lax.dot_general(
        w_ref[...], h, (((0,), (0,)), ((), ())),
        preferred_element_type=jnp.float32)


def _stage3_body(z_ref, y_ref, dinv_ref, b_ref, w3_ref, out_ref):
    dinv = dinv_ref[...]
    h = jnp.tanh(dinv * (z_ref[...] + y_ref[...]) + b_ref[...])
    out_ref[...] = dinv * jnp.sum(h * w3_ref[...], axis=0, keepdims=True)


def _stage4_body(zp_ref, y_ref, dinv_ref, b3_ref, w4_ref, b4_ref, out_ref):
    z = jnp.sum(zp_ref[...], axis=0, keepdims=True)
    h = jnp.tanh(dinv_ref[...] * (z + y_ref[...]) + b3_ref[...])
    out_ref[...] = jnp.sum(h * w4_ref[...], axis=1, keepdims=True) + b4_ref[...]


def kernel(x, edge_index, W1, b1, W2, b2, W3, b3, W4, b4):
    n, f1 = x.shape
    e = edge_index.shape[1]
    h1 = W1.shape[1]
    h2 = W2.shape[1]
    row = edge_index[0]
    col = edge_index[1]
    chunk = 8000

    cnt = _make_count(n, e)(col)

    yt1, dinv = pl.pallas_call(
        _stage1_body,
        out_shape=[
            jax.ShapeDtypeStruct((h1, n), jnp.float32),
            jax.ShapeDtypeStruct((1, n), jnp.float32),
        ],
    )(W1, x, cnt)

    z1 = _make_prop(n, e, h1, chunk)(yt1, row, col)

    yt2 = pl.pallas_call(
        _mid_body,
        out_shape=jax.ShapeDtypeStruct((h2, n), jnp.float32),
    )(z1, yt1, dinv, b1.reshape(-1, 1), W2)

    z2 = _make_prop(n, e, h2, chunk)(yt2, row, col)

    yt3 = pl.pallas_call(
        _stage3_body,
        out_shape=jax.ShapeDtypeStruct((1, n), jnp.float32),
    )(z2, yt2, dinv, b2.reshape(-1, 1), W3)

    z3p = _make_prop1(n, e)(yt3, row, col)

    out = pl.pallas_call(
        _stage4_body,
        out_shape=jax.ShapeDtypeStruct((1, 1), jnp.float32),
    )(z3p, yt3, dinv, b3.reshape(1, 1), W4.reshape(1, -1), b4.reshape(1, 1))

    return out.reshape(1)


# speedup vs baseline: 21.9793x; 1.7953x over previous
import functools

import jax
import jax.numpy as jnp
from jax import lax
from jax.experimental import pallas as pl
from jax.experimental.pallas import tpu as pltpu
from jax.experimental.pallas import tpu_sc as plsc

_NC = 2
_NS = 16
_NW = _NC * _NS
_L = 16


def _mesh():
    return plsc.VectorSubcoreMesh(
        core_axis_name="c", subcore_axis_name="s",
        num_cores=_NC, num_subcores=_NS)


_SC_PARAMS = pltpu.CompilerParams(needs_layout_passes=False)


def _wid():
    return lax.axis_index("s") * _NC + lax.axis_index("c")


def _zero_loop(refs, n):
    zeros = jnp.zeros((_L,), jnp.float32)

    def body(i, carry):
        for r in refs:
            r[pl.ds(i * _L, _L)] = zeros
        return carry

    lax.fori_loop(0, n // _L, body, 0)


@functools.lru_cache(maxsize=None)
def _make_count(n, e):
    e_per = e // _NW

    @functools.partial(
        pl.kernel,
        mesh=_mesh(),
        compiler_params=_SC_PARAMS,
        out_type=jax.ShapeDtypeStruct((_NW, n), jnp.float32),
        scratch_types=[
            pltpu.VMEM((e_per,), jnp.int32),
            pltpu.VMEM((n,), jnp.float32),
        ],
    )
    def count(col_hbm, out_hbm, cb, ob):
        w = _wid()
        pltpu.sync_copy(col_hbm.at[pl.ds(w * e_per, e_per)], cb)
        _zero_loop([ob], n)
        ones = jnp.ones((_L,), jnp.float32)

        def step(j, carry):
            idx = cb[pl.ds(j * _L, _L)]
            plsc.addupdate_scatter(ob, [idx], ones)
            return carry

        lax.fori_loop(0, e_per // _L, step, 0)
        pltpu.sync_copy(ob, out_hbm.at[w])

    return count


@functools.lru_cache(maxsize=None)
def _make_prop1(n, e):
    e_per = e // _NW

    @functools.partial(
        pl.kernel,
        mesh=_mesh(),
        compiler_params=_SC_PARAMS,
        out_type=jax.ShapeDtypeStruct((_NW, n), jnp.float32),
        scratch_types=[
            pltpu.VMEM((n,), jnp.float32),
            pltpu.VMEM((e_per,), jnp.int32),
            pltpu.VMEM((e_per,), jnp.int32),
            pltpu.VMEM((n,), jnp.float32),
        ],
    )
    def prop1(y_hbm, row_hbm, col_hbm, out_hbm, yv, rb, cb, ob):
        w = _wid()
        pltpu.sync_copy(y_hbm.at[0], yv)
        pltpu.sync_copy(row_hbm.at[pl.ds(w * e_per, e_per)], rb)
        pltpu.sync_copy(col_hbm.at[pl.ds(w * e_per, e_per)], cb)
        _zero_loop([ob], n)

        u = 5

        def step(j, carry):
            base = j * (_L * u)
            gathered = []
            for k in range(u):
                r = rb[pl.ds(base + k * _L, _L)]
                c = cb[pl.ds(base + k * _L, _L)]
                gathered.append((c, plsc.load_gather(yv, [r])))
            for c, v in gathered:
                plsc.addupdate_scatter(ob, [c], v)
            return carry

        lax.fori_loop(0, e_per // (_L * u), step, 0)
        pltpu.sync_copy(ob, out_hbm.at[w])

    return prop1


@functools.lru_cache(maxsize=None)
def _make_prop(n, e, dfull, chunk):
    d = dfull // _NW
    nchunks = e // chunk
    unroll = 8 // d if d < 8 else 1
    assert chunk % (_L * unroll) == 0
    scratch = (
        [pltpu.VMEM((n,), jnp.float32) for _ in range(2 * d)]
        + [pltpu.VMEM((chunk,), jnp.int32) for _ in range(2)]
    )

    @functools.partial(
        pl.kernel,
        mesh=_mesh(),
        compiler_params=_SC_PARAMS,
        out_type=jax.ShapeDtypeStruct((dfull, n), jnp.float32),
        scratch_types=scratch,
    )
    def prop(y_hbm, row_hbm, col_hbm, out_hbm, *bufs):
        yb = bufs[:d]
        zb = bufs[d:2 * d]
        rb = bufs[2 * d]
        cb = bufs[2 * d + 1]
        w = _wid()
        f0 = w * d
        for f in range(d):
            pltpu.sync_copy(y_hbm.at[f0 + f], yb[f])
        _zero_loop(list(zb), n)

        def chunk_body(k, carry):
            pltpu.sync_copy(row_hbm.at[pl.ds(k * chunk, chunk)], rb)
            pltpu.sync_copy(col_hbm.at[pl.ds(k * chunk, chunk)], cb)

            def step(j, c2):
                base = j * (_L * unroll)
                gathered = []
                for k in range(unroll):
                    r = rb[pl.ds(base + k * _L, _L)]
                    c = cb[pl.ds(base + k * _L, _L)]
                    for f in range(d):
                        gathered.append((f, c, plsc.load_gather(yb[f], [r])))
                for f, c, v in gathered:
                    plsc.addupdate_scatter(zb[f], [c], v)
                return c2

            lax.fori_loop(0, chunk // (_L * unroll), step, 0)
            return carry

        lax.fori_loop(0, nchunks, chunk_body, 0)
        for f in range(d):
            pltpu.sync_copy(zb[f], out_hbm.at[f0 + f])

    return prop


def _stage1_body(w1_ref, x_ref, cnt_ref, yt_ref, dinv_ref):
    deg = 1.0 + jnp.sum(cnt_ref[...], axis=0, keepdims=True)
    dinv = lax.rsqrt(deg)
    dinv_ref[...] = dinv
    yt_ref[...] = dinv * lax.dot_general(
        w1_ref[...], x_ref[...], (((0,), (1,)), ((), ())),
        preferred_element_type=jnp.float32)


def _mid_body(z_ref, y_ref, dinv_ref, b_ref, w_ref, out_ref):
    dinv = dinv_ref[...]
    h = jnp.tanh(dinv * (z_ref[...] + y_ref[...]) + b_ref[...])
    out_ref[...] = dinv * lax.dot_general(
        w_ref[...], h, (((0,), (0,)), ((), ())),
        preferred_element_type=jnp.float32)


def _stage3_body(z_ref, y_ref, dinv_ref, b_ref, w3_ref, out_ref):
    dinv = dinv_ref[...]
    h = jnp.tanh(dinv * (z_ref[...] + y_ref[...]) + b_ref[...])
    out_ref[...] = dinv * jnp.sum(h * w3_ref[...], axis=0, keepdims=True)


def _stage4_body(zp_ref, y_ref, dinv_ref, b3_ref, w4_ref, b4_ref, out_ref):
    z = jnp.sum(zp_ref[...], axis=0, keepdims=True)
    h = jnp.tanh(dinv_ref[...] * (z + y_ref[...]) + b3_ref[...])
    out_ref[...] = jnp.sum(h * w4_ref[...], axis=1, keepdims=True) + b4_ref[...]


def kernel(x, edge_index, W1, b1, W2, b2, W3, b3, W4, b4):
    n, f1 = x.shape
    e = edge_index.shape[1]
    h1 = W1.shape[1]
    h2 = W2.shape[1]
    row = edge_index[0]
    col = edge_index[1]
    chunk = 8000

    cnt = _make_count(n, e)(col)

    yt1, dinv = pl.pallas_call(
        _stage1_body,
        out_shape=[
            jax.ShapeDtypeStruct((h1, n), jnp.float32),
            jax.ShapeDtypeStruct((1, n), jnp.float32),
        ],
    )(W1, x, cnt)

    z1 = _make_prop(n, e, h1, chunk)(yt1, row, col)

    yt2 = pl.pallas_call(
        _mid_body,
        out_shape=jax.ShapeDtypeStruct((h2, n), jnp.float32),
    )(z1, yt1, dinv, b1.reshape(-1, 1), W2)

    z2 = _make_prop(n, e, h2, chunk)(yt2, row, col)

    yt3 = pl.pallas_call(
        _stage3_body,
        out_shape=jax.ShapeDtypeStruct((1, n), jnp.float32),
    )(z2, yt2, dinv, b2.reshape(-1, 1), W3)

    z3p = _make_prop1(n, e)(yt3, row, col)

    out = pl.pallas_call(
        _stage4_body,
        out_shape=jax.ShapeDtypeStruct((1, 1), jnp.float32),
    )(z3p, yt3, dinv, b3.reshape(1, 1), W4.reshape(1, -1), b4.reshape(1, 1))

    return out.reshape(1)
